# Initial kernel scaffold; baseline (speedup 1.0000x reference)
import jax
import jax.numpy as jnp
from jax.experimental import pallas as pl


def kernel(x, codebook):
    raise NotImplementedError("write your pallas kernel here")


# speedup vs baseline: 1.9931x; 1.9931x over previous
import functools

import jax
import jax.numpy as jnp
from jax.experimental import pallas as pl
from jax.experimental.pallas import tpu as pltpu

N_COMPONENTS = 1024
EMBEDDING_DIM = 64
BETA = 0.25

_BN = 512


def _vq_block(x_ref, cb_ref, q_ref, sc_ref, loss_ref):
    i = pl.program_id(0)
    x = x_ref[...]
    cb = cb_ref[...]
    sim = jnp.dot(x, cb, preferred_element_type=jnp.float32)
    x2 = jnp.sum(x * x, axis=1, keepdims=True)
    c2 = jnp.sum(cb * cb, axis=0, keepdims=True)
    d = x2 + c2 - 2.0 * sim
    inv = (1.0 / d) ** 2
    sc_ref[...] = inv / jnp.sum(inv, axis=1, keepdims=True)
    idx = jnp.argmin(d, axis=1)
    onehot = (jax.lax.broadcasted_iota(jnp.int32, d.shape, 1)
              == idx[:, None]).astype(jnp.float32)
    q = jax.lax.dot_general(onehot, cb, (((1,), (1,)), ((), ())),
                            preferred_element_type=jnp.float32)
    q_ref[...] = q
    diff = q - x
    part = jnp.sum(diff * diff).reshape(1, 1)

    @pl.when(i == 0)
    def _init():
        loss_ref[...] = jnp.zeros((1, 1), jnp.float32)

    loss_ref[...] += part


@jax.jit
def kernel(x, codebook):
    input_shape = x.shape
    n = x.shape[0] * x.shape[1]
    xf = x.reshape(n, EMBEDDING_DIM)
    grid = n // _BN
    q, sc, loss = pl.pallas_call(
        _vq_block,
        grid=(grid,),
        in_specs=[
            pl.BlockSpec((_BN, EMBEDDING_DIM), lambda i: (i, 0)),
            pl.BlockSpec((EMBEDDING_DIM, N_COMPONENTS), lambda i: (0, 0)),
        ],
        out_specs=[
            pl.BlockSpec((_BN, EMBEDDING_DIM), lambda i: (i, 0)),
            pl.BlockSpec((_BN, N_COMPONENTS), lambda i: (i, 0)),
            pl.BlockSpec((1, 1), lambda i: (0, 0)),
        ],
        out_shape=[
            jax.ShapeDtypeStruct((n, EMBEDDING_DIM), jnp.float32),
            jax.ShapeDtypeStruct((n, N_COMPONENTS), jnp.float32),
            jax.ShapeDtypeStruct((1, 1), jnp.float32),
        ],
    )(xf, codebook)
    vq_loss = (1.0 + BETA) * loss[0, 0] / (n * EMBEDDING_DIM)
    return q.reshape(input_shape), sc, vq_loss


# speedup vs baseline: 2.0276x; 1.0174x over previous
import functools

import jax
import jax.numpy as jnp
from jax.experimental import pallas as pl
from jax.experimental.pallas import tpu as pltpu

N_COMPONENTS = 1024
EMBEDDING_DIM = 64
BETA = 0.25

_BN = 512


def _vq_block(x_ref, cb_ref, q_ref, sc_ref, loss_ref):
    i = pl.program_id(0)
    x = x_ref[...]
    cb = cb_ref[...]
    sim = jnp.dot(x, cb, preferred_element_type=jnp.float32)
    x2 = jnp.sum(x * x, axis=1, keepdims=True)
    c2 = jnp.sum(cb * cb, axis=0, keepdims=True)
    d = x2 + c2 - 2.0 * sim
    r = 1.0 / d
    inv = r * r
    rows = jnp.sum(inv, axis=1, keepdims=True)
    sc_ref[...] = inv * (1.0 / rows)
    idx = jnp.argmin(d, axis=1)
    onehot = (jax.lax.broadcasted_iota(jnp.int32, d.shape, 1)
              == idx[:, None]).astype(jnp.float32)
    q = jax.lax.dot_general(onehot, cb, (((1,), (1,)), ((), ())),
                            preferred_element_type=jnp.float32)
    q_ref[...] = q
    part = jnp.sum(jnp.min(d, axis=1)).reshape(1, 1)

    @pl.when(i == 0)
    def _init():
        loss_ref[...] = jnp.zeros((1, 1), jnp.float32)

    loss_ref[...] += part


@jax.jit
def kernel(x, codebook):
    input_shape = x.shape
    n = x.shape[0] * x.shape[1]
    xf = x.reshape(n, EMBEDDING_DIM)
    grid = n // _BN
    q, sc, loss = pl.pallas_call(
        _vq_block,
        grid=(grid,),
        in_specs=[
            pl.BlockSpec((_BN, EMBEDDING_DIM), lambda i: (i, 0)),
            pl.BlockSpec((EMBEDDING_DIM, N_COMPONENTS), lambda i: (0, 0)),
        ],
        out_specs=[
            pl.BlockSpec((_BN, EMBEDDING_DIM), lambda i: (i, 0)),
            pl.BlockSpec((_BN, N_COMPONENTS), lambda i: (i, 0)),
            pl.BlockSpec((1, 1), lambda i: (0, 0)),
        ],
        out_shape=[
            jax.ShapeDtypeStruct((n, EMBEDDING_DIM), jnp.float32),
            jax.ShapeDtypeStruct((n, N_COMPONENTS), jnp.float32),
            jax.ShapeDtypeStruct((1, 1), jnp.float32),
        ],
    )(xf, codebook)
    vq_loss = (1.0 + BETA) * loss[0, 0] / (n * EMBEDDING_DIM)
    return q.reshape(input_shape), sc, vq_loss


# speedup vs baseline: 2.0632x; 1.0175x over previous
import functools

import jax
import jax.numpy as jnp
from jax.experimental import pallas as pl
from jax.experimental.pallas import tpu as pltpu

N_COMPONENTS = 1024
EMBEDDING_DIM = 64
BETA = 0.25

_BN = 1024


def _vq_block(x_ref, cb_ref, q_ref, sc_ref, loss_ref, c2_ref):
    i = pl.program_id(0)
    x = x_ref[...]
    cb = cb_ref[...]

    @pl.when(i == 0)
    def _prep():
        c2_ref[...] = jnp.sum(cb * cb, axis=0, keepdims=True)

    sim = jnp.dot(x, cb, preferred_element_type=jnp.float32)
    x2 = jnp.sum(x * x, axis=1, keepdims=True)
    d = x2 + c2_ref[...] - 2.0 * sim
    r = 1.0 / d
    inv = r * r
    rows = jnp.sum(inv, axis=1, keepdims=True)
    imax = jnp.max(inv, axis=1, keepdims=True)
    sc_ref[...] = inv * (1.0 / rows)
    idx = jnp.argmin(d, axis=1)
    onehot = (jax.lax.broadcasted_iota(jnp.int32, d.shape, 1)
              == idx[:, None]).astype(jnp.float32)
    q = jax.lax.dot_general(onehot, cb, (((1,), (1,)), ((), ())),
                            preferred_element_type=jnp.float32)
    q_ref[...] = q
    part = jnp.sum(jax.lax.rsqrt(imax)).reshape(1, 1)

    @pl.when(i == 0)
    def _init():
        loss_ref[...] = jnp.zeros((1, 1), jnp.float32)

    loss_ref[...] += part


@jax.jit
def kernel(x, codebook):
    input_shape = x.shape
    n = x.shape[0] * x.shape[1]
    xf = x.reshape(n, EMBEDDING_DIM)
    grid = n // _BN
    q, sc, loss = pl.pallas_call(
        _vq_block,
        grid=(grid,),
        in_specs=[
            pl.BlockSpec((_BN, EMBEDDING_DIM), lambda i: (i, 0)),
            pl.BlockSpec((EMBEDDING_DIM, N_COMPONENTS), lambda i: (0, 0)),
        ],
        out_specs=[
            pl.BlockSpec((_BN, EMBEDDING_DIM), lambda i: (i, 0)),
            pl.BlockSpec((_BN, N_COMPONENTS), lambda i: (i, 0)),
            pl.BlockSpec((1, 1), lambda i: (0, 0)),
        ],
        out_shape=[
            jax.ShapeDtypeStruct((n, EMBEDDING_DIM), jnp.float32),
            jax.ShapeDtypeStruct((n, N_COMPONENTS), jnp.float32),
            jax.ShapeDtypeStruct((1, 1), jnp.float32),
        ],
        scratch_shapes=[pltpu.VMEM((1, N_COMPONENTS), jnp.float32)],
    )(xf, codebook)
    vq_loss = (1.0 + BETA) * loss[0, 0] / (n * EMBEDDING_DIM)
    return q.reshape(input_shape), sc, vq_loss
